# baseline (device time: 154755 ns/iter reference)
import jax
import jax.numpy as jnp
from jax import lax
from jax.experimental import pallas as pl
from jax.experimental.pallas import tpu as pltpu

N = 8
B, S, D = 4, 256, 4096
H, Dh, Dr = 32, 128, 64
HL = H // N
KC = HL * Dh
QRC = HL * Dr
M = B * S
RC = M // N
SCALE = (Dh + Dr) ** -0.5
BF = jnp.bfloat16
VMEM = pltpu.VMEM


def _kvq_body(x_ref, wdkv_ref, wukb_ref, wuvb_ref, wq_ref, wqr_ref, wkr_ref,
              q_ref, qr_ref, kr_ref, kj_ref, vj_ref,
              c_s, call_s, wka_s, wva_s, cs, cr, wks, wkrv, wvs, wvrv):
    i = lax.axis_index("i")
    blk = (i + 1) % N

    c_s[...] = jnp.dot(x_ref[...], wdkv_ref[...],
                       preferred_element_type=jnp.float32).astype(BF)

    bsem = pltpu.get_barrier_semaphore()
    for d in range(1, N):
        pl.semaphore_signal(bsem, inc=1, device_id=((i + d) % N,),
                            device_id_type=pl.DeviceIdType.MESH)
    pl.semaphore_wait(bsem, N - 1)

    dc = c_s.shape[1]
    descs = []
    for d in range(1, N):
        tgt = (i + d) % N
        blk_tgt = (tgt + 1) % N
        for src, dst, ssem, rsem in (
            (c_s, call_s.at[:, pl.ds((d - 1) * dc, dc)], cs, cr),
            (wukb_ref.at[blk_tgt],
             wka_s.at[pl.ds((d - 1) * dc, dc), :], wks, wkrv),
            (wuvb_ref.at[blk_tgt],
             wva_s.at[pl.ds((d - 1) * dc, dc), :], wvs, wvrv),
        ):
            rdma = pltpu.make_async_remote_copy(
                src_ref=src,
                dst_ref=dst,
                send_sem=ssem.at[d - 1],
                recv_sem=rsem.at[d - 1],
                device_id=(tgt,),
                device_id_type=pl.DeviceIdType.MESH,
            )
            rdma.start()
            descs.append(rdma)

    call_s[:, (N - 1) * dc:] = c_s[...]
    wka_s[(N - 1) * dc:, :] = wukb_ref[blk]
    wva_s[(N - 1) * dc:, :] = wuvb_ref[blk]
    x = x_ref[...]
    q_ref[...] = jnp.dot(x, wq_ref[...],
                         preferred_element_type=jnp.float32).astype(BF)
    qr_ref[...] = jnp.dot(x, wqr_ref[...],
                          preferred_element_type=jnp.float32).astype(BF)
    kr_ref[...] = jnp.dot(x, wkr_ref[...],
                          preferred_element_type=jnp.float32).astype(BF)

    for rdma in descs:
        rdma.wait()

    cfull = call_s[...]
    kj_ref[...] = jnp.dot(cfull, wka_s[...],
                          preferred_element_type=jnp.float32).astype(BF)
    vj_ref[...] = jnp.dot(cfull, wva_s[...],
                          preferred_element_type=jnp.float32).astype(BF)


def _kvq(xb, wdkv, wukb, wuvb, wq_j, wqr_j, wkr):
    dc = wdkv.shape[1]
    return pl.pallas_call(
        _kvq_body,
        out_shape=[jax.ShapeDtypeStruct((M, KC), BF),
                   jax.ShapeDtypeStruct((M, QRC), BF),
                   jax.ShapeDtypeStruct((M, Dr), BF),
                   jax.ShapeDtypeStruct((M, KC), BF),
                   jax.ShapeDtypeStruct((M, KC), BF)],
        in_specs=[pl.BlockSpec(memory_space=VMEM)] * 7,
        out_specs=[pl.BlockSpec(memory_space=VMEM)] * 5,
        scratch_shapes=[
            VMEM((M, dc), BF),
            VMEM((M, N * dc), BF),
            VMEM((N * dc, KC), BF),
            VMEM((N * dc, KC), BF),
            pltpu.SemaphoreType.DMA((N - 1,)),
            pltpu.SemaphoreType.DMA((N - 1,)),
            pltpu.SemaphoreType.DMA((N - 1,)),
            pltpu.SemaphoreType.DMA((N - 1,)),
            pltpu.SemaphoreType.DMA((N - 1,)),
            pltpu.SemaphoreType.DMA((N - 1,)),
        ],
        compiler_params=pltpu.CompilerParams(collective_id=0),
    )(xb, wdkv, wukb, wuvb, wq_j, wqr_j, wkr)


def _attn_body(q_ref, k_ref, v_ref, qr_ref, kr_ref, o_ref):
    kr = kr_ref[...]
    nums = (((1,), (1,)), ((), ()))
    for h in range(HL):
        hd = slice(h * Dh, (h + 1) * Dh)
        hr = slice(h * Dr, (h + 1) * Dr)
        s = (lax.dot_general(q_ref[:, hd], k_ref[:, hd], nums,
                             preferred_element_type=jnp.float32)
             + lax.dot_general(qr_ref[:, hr], kr, nums,
                               preferred_element_type=jnp.float32)) * SCALE
        m = jnp.max(s, axis=-1, keepdims=True)
        e = jnp.exp(s - m)
        p = (e / jnp.sum(e, axis=-1, keepdims=True)).astype(BF)
        o_ref[:, hd] = jnp.dot(p, v_ref[:, hd],
                               preferred_element_type=jnp.float32).astype(BF)


def _attention(q, kj, vj, qr, kr):
    return pl.pallas_call(
        _attn_body,
        grid=(B,),
        out_shape=jax.ShapeDtypeStruct((M, KC), BF),
        in_specs=[
            pl.BlockSpec((S, KC), lambda b: (b, 0)),
            pl.BlockSpec((S, KC), lambda b: (b, 0)),
            pl.BlockSpec((S, KC), lambda b: (b, 0)),
            pl.BlockSpec((S, QRC), lambda b: (b, 0)),
            pl.BlockSpec((S, Dr), lambda b: (b, 0)),
        ],
        out_specs=pl.BlockSpec((S, KC), lambda b: (b, 0)),
    )(q, kj, vj, qr, kr)


RH = RC // 2


def _outar_body(q_ref, kj_ref, vj_ref, qr_ref, kr_ref, wo_ref, out_ref,
                buf_r, buf_l,
                rs_r_s, rs_r_r, rs_l_s, rs_l_r,
                ag_s, ag_r):
    i = lax.axis_index("i")
    left = (i - 1) % N
    right = (i + 1) % N
    bsem = pltpu.get_barrier_semaphore()
    for nbr in (left, right):
        pl.semaphore_signal(bsem, inc=1, device_id=(nbr,),
                            device_id_type=pl.DeviceIdType.MESH)
    pl.semaphore_wait(bsem, 2)

    wo = wo_ref[...]
    nums = (((1,), (1,)), ((), ()))

    def att_rows(sub):
        b0 = (sub // (S // RH)) * S
        qs = q_ref[pl.ds(sub * RH, RH), :]
        qrs = qr_ref[pl.ds(sub * RH, RH), :]
        kb = kj_ref[pl.ds(b0, S), :]
        vb = vj_ref[pl.ds(b0, S), :]
        krb = kr_ref[pl.ds(b0, S), :]
        ohs = []
        for h in range(HL):
            hd = slice(h * Dh, (h + 1) * Dh)
            hr = slice(h * Dr, (h + 1) * Dr)
            s = (lax.dot_general(qs[:, hd], kb[:, hd], nums,
                                 preferred_element_type=jnp.float32)
                 + lax.dot_general(qrs[:, hr], krb, nums,
                                   preferred_element_type=jnp.float32)) * SCALE
            m = jnp.max(s, axis=-1, keepdims=True)
            e = jnp.exp(s - m)
            p = (e / jnp.sum(e, axis=-1, keepdims=True)).astype(BF)
            ohs.append(jnp.dot(p, vb[:, hd],
                               preferred_element_type=jnp.float32).astype(BF))
        return jnp.concatenate(ohs, axis=1)

    def part2(sub_r, sub_l):
        o2 = jnp.concatenate([att_rows(sub_r), att_rows(sub_l)], axis=0)
        g = jnp.dot(o2, wo, preferred_element_type=jnp.float32)
        return g[:RH], g[RH:]

    pending = []
    NCH = 4
    CH = D // NCH
    sub_t = 2 * ((i + 1) % N)
    sub_b = 2 * ((i - 1) % N) + 1

    def rs_start(buf, ch, hop, dev, ssem, rsem):
        src = N - 1 if hop == 0 else hop - 1
        sl = pl.ds(ch * CH, CH)
        rd = pltpu.make_async_remote_copy(
            src_ref=buf.at[src, :, sl], dst_ref=buf.at[hop, :, sl],
            send_sem=ssem.at[ch, hop], recv_sem=rsem.at[ch, hop],
            device_id=(dev,), device_id_type=pl.DeviceIdType.MESH)
        rd.start()
        pending.append(rd)
        return rd

    own_r, own_l = part2(2 * i, 2 * i + 1)
    buf_r[N - 1] = own_r.astype(BF)
    buf_l[N - 1] = own_l.astype(BF)
    rds_r = [rs_start(buf_r, ch, 0, right, rs_r_s, rs_r_r)
             for ch in range(NCH)]
    rds_l = [rs_start(buf_l, ch, 0, left, rs_l_s, rs_l_r)
             for ch in range(NCH)]
    nxt_r, nxt_l = part2(2 * ((i - 1) % N), 2 * ((i + 1) % N) + 1)
    for s in range(N - 1):
        for ch in range(NCH):
            sl = slice(ch * CH, (ch + 1) * CH)
            rds_r[ch].wait_recv()
            rds_l[ch].wait_recv()
            if s < N - 2:
                buf_r[s, :, sl] = (buf_r[s, :, sl] + nxt_r[:, sl]).astype(BF)
                buf_l[s, :, sl] = (buf_l[s, :, sl] + nxt_l[:, sl]).astype(BF)
                rds_r[ch] = rs_start(buf_r, ch, s + 1, right, rs_r_s, rs_r_r)
                rds_l[ch] = rs_start(buf_l, ch, s + 1, left, rs_l_s, rs_l_r)
                if ch == 0:
                    nnr, nnl = part2(2 * ((i - s - 2) % N),
                                     2 * ((i + s + 2) % N) + 1)
            else:
                out_ref[sub_t, :, sl] = (buf_r[s, :, sl]
                                         + nxt_r[:, sl]).astype(BF)
                out_ref[sub_b, :, sl] = (buf_l[s, :, sl]
                                         + nxt_l[:, sl]).astype(BF)
        if s < N - 2:
            nxt_r, nxt_l = nnr, nnl

    def PX(d):
        return d ^ 1

    def PY(d):
        return 4 * (d // 4) + (3 - d % 4)

    def PZ(d):
        return d ^ 4

    tstarts = (0, 1280, 2688)
    twidths = (1280, 1408, 1408)
    orders = ((PX, PY, PZ), (PY, PZ, PX), (PZ, PX, PY))
    held = [[i], [i], [i]]
    ctr = [0, 0, 0]
    for p in range(3):
        phase_descs = []
        for t in range(3):
            P = orders[t][p]
            prt = P(i)
            for d in held[t]:
                for sub in (2 * ((d + 1) % N), 2 * ((d - 1) % N) + 1):
                    piece = out_ref.at[sub, :, pl.ds(tstarts[t], twidths[t])]
                    rd = pltpu.make_async_remote_copy(
                        src_ref=piece, dst_ref=piece,
                        send_sem=ag_s.at[t, ctr[t]],
                        recv_sem=ag_r.at[t, ctr[t]],
                        device_id=(prt,),
                        device_id_type=pl.DeviceIdType.MESH)
                    rd.start()
                    ctr[t] += 1
                    phase_descs.append(rd)
                    pending.append(rd)
            held[t] = held[t] + [P(d) for d in held[t]]
        for rd in phase_descs:
            rd.wait_recv()

    for rd in pending:
        rd.wait_send()


def _outar(q, kj, vj, qr, kr, wo_j):
    return pl.pallas_call(
        _outar_body,
        out_shape=jax.ShapeDtypeStruct((2 * N, RH, D), BF),
        in_specs=[pl.BlockSpec(memory_space=VMEM)] * 6,
        out_specs=pl.BlockSpec(memory_space=VMEM),
        scratch_shapes=[
            VMEM((N, RH, D), BF),
            VMEM((N, RH, D), BF),
            pltpu.SemaphoreType.DMA((4, N - 1)),
            pltpu.SemaphoreType.DMA((4, N - 1)),
            pltpu.SemaphoreType.DMA((4, N - 1)),
            pltpu.SemaphoreType.DMA((4, N - 1)),
            pltpu.SemaphoreType.DMA((3, 14)),
            pltpu.SemaphoreType.DMA((3, 14)),
        ],
        compiler_params=pltpu.CompilerParams(collective_id=1),
    )(q, kj, vj, qr, kr, wo_j)


def kernel(x, Wdkv, Wuk, Wuv, Wq, Wqr, Wkr, Wo):
    i = lax.axis_index("i")
    blk = (i + 1) % N

    xb = x.reshape(M, D).astype(BF)
    dc = Wdkv.shape[1]
    wukb = Wuk.astype(BF).reshape(dc, N, KC).transpose(1, 0, 2)
    wuvb = Wuv.astype(BF).reshape(dc, N, KC).transpose(1, 0, 2)

    wq_j = lax.dynamic_slice(Wq, (0, blk * KC), (D, KC)).astype(BF)
    wqr_j = lax.dynamic_slice(Wqr, (0, blk * QRC), (D, QRC)).astype(BF)
    wo_j = lax.dynamic_slice(Wo, (blk * KC, 0), (KC, D)).astype(BF)
    q, qr, kr, kj, vj = _kvq(xb, Wdkv.astype(BF), wukb, wuvb,
                             wq_j, wqr_j, Wkr.astype(BF))

    out16 = _outar(q, kj, vj, qr, kr, wo_j)
    return out16.reshape(M, D).reshape(B, S, D)


# device time: 153093 ns/iter; 1.0109x vs baseline; 1.0109x over previous
import jax
import jax.numpy as jnp
from jax import lax
from jax.experimental import pallas as pl
from jax.experimental.pallas import tpu as pltpu

N = 8
B, S, D = 4, 256, 4096
H, Dh, Dr = 32, 128, 64
HL = H // N
KC = HL * Dh
QRC = HL * Dr
M = B * S
RC = M // N
SCALE = (Dh + Dr) ** -0.5
BF = jnp.bfloat16
VMEM = pltpu.VMEM


def _kvq_body(x_ref, wdkv_ref, wukb_ref, wuvb_ref, wq_ref, wqr_ref, wkr_ref,
              q_ref, qr_ref, kr_ref, kj_ref, vj_ref,
              c_s, call_s, wka_s, wva_s, cs, cr, wks, wkrv, wvs, wvrv):
    i = lax.axis_index("i")
    blk = (i + 1) % N

    c_s[...] = jnp.dot(x_ref[...], wdkv_ref[...],
                       preferred_element_type=jnp.float32).astype(BF)

    bsem = pltpu.get_barrier_semaphore()
    for d in range(1, N):
        pl.semaphore_signal(bsem, inc=1, device_id=((i + d) % N,),
                            device_id_type=pl.DeviceIdType.MESH)
    pl.semaphore_wait(bsem, N - 1)

    dc = c_s.shape[1]
    descs = []
    for d in range(1, N):
        tgt = (i + d) % N
        blk_tgt = (tgt + 1) % N
        for src, dst, ssem, rsem in (
            (c_s, call_s.at[:, pl.ds((d - 1) * dc, dc)], cs, cr),
            (wukb_ref.at[blk_tgt],
             wka_s.at[pl.ds((d - 1) * dc, dc), :], wks, wkrv),
            (wuvb_ref.at[blk_tgt],
             wva_s.at[pl.ds((d - 1) * dc, dc), :], wvs, wvrv),
        ):
            rdma = pltpu.make_async_remote_copy(
                src_ref=src,
                dst_ref=dst,
                send_sem=ssem.at[d - 1],
                recv_sem=rsem.at[d - 1],
                device_id=(tgt,),
                device_id_type=pl.DeviceIdType.MESH,
            )
            rdma.start()
            descs.append(rdma)

    call_s[:, (N - 1) * dc:] = c_s[...]
    wka_s[(N - 1) * dc:, :] = wukb_ref[blk]
    wva_s[(N - 1) * dc:, :] = wuvb_ref[blk]
    x = x_ref[...]
    q_ref[...] = jnp.dot(x, wq_ref[...],
                         preferred_element_type=jnp.float32).astype(BF)
    qr_ref[...] = jnp.dot(x, wqr_ref[...],
                          preferred_element_type=jnp.float32).astype(BF)
    kr_ref[...] = jnp.dot(x, wkr_ref[...],
                          preferred_element_type=jnp.float32).astype(BF)

    for rdma in descs:
        rdma.wait()

    cfull = call_s[...]
    kj_ref[...] = jnp.dot(cfull, wka_s[...],
                          preferred_element_type=jnp.float32).astype(BF)
    vj_ref[...] = jnp.dot(cfull, wva_s[...],
                          preferred_element_type=jnp.float32).astype(BF)


def _kvq(xb, wdkv, wukb, wuvb, wq_j, wqr_j, wkr):
    dc = wdkv.shape[1]
    return pl.pallas_call(
        _kvq_body,
        out_shape=[jax.ShapeDtypeStruct((M, KC), BF),
                   jax.ShapeDtypeStruct((M, QRC), BF),
                   jax.ShapeDtypeStruct((M, Dr), BF),
                   jax.ShapeDtypeStruct((M, KC), BF),
                   jax.ShapeDtypeStruct((M, KC), BF)],
        in_specs=[pl.BlockSpec(memory_space=VMEM)] * 7,
        out_specs=[pl.BlockSpec(memory_space=VMEM)] * 5,
        scratch_shapes=[
            VMEM((M, dc), BF),
            VMEM((M, N * dc), BF),
            VMEM((N * dc, KC), BF),
            VMEM((N * dc, KC), BF),
            pltpu.SemaphoreType.DMA((N - 1,)),
            pltpu.SemaphoreType.DMA((N - 1,)),
            pltpu.SemaphoreType.DMA((N - 1,)),
            pltpu.SemaphoreType.DMA((N - 1,)),
            pltpu.SemaphoreType.DMA((N - 1,)),
            pltpu.SemaphoreType.DMA((N - 1,)),
        ],
        compiler_params=pltpu.CompilerParams(collective_id=0),
    )(xb, wdkv, wukb, wuvb, wq_j, wqr_j, wkr)


def _attn_body(q_ref, k_ref, v_ref, qr_ref, kr_ref, o_ref):
    kr = kr_ref[...]
    nums = (((1,), (1,)), ((), ()))
    for h in range(HL):
        hd = slice(h * Dh, (h + 1) * Dh)
        hr = slice(h * Dr, (h + 1) * Dr)
        s = (lax.dot_general(q_ref[:, hd], k_ref[:, hd], nums,
                             preferred_element_type=jnp.float32)
             + lax.dot_general(qr_ref[:, hr], kr, nums,
                               preferred_element_type=jnp.float32)) * SCALE
        m = jnp.max(s, axis=-1, keepdims=True)
        e = jnp.exp(s - m)
        p = (e / jnp.sum(e, axis=-1, keepdims=True)).astype(BF)
        o_ref[:, hd] = jnp.dot(p, v_ref[:, hd],
                               preferred_element_type=jnp.float32).astype(BF)


def _attention(q, kj, vj, qr, kr):
    return pl.pallas_call(
        _attn_body,
        grid=(B,),
        out_shape=jax.ShapeDtypeStruct((M, KC), BF),
        in_specs=[
            pl.BlockSpec((S, KC), lambda b: (b, 0)),
            pl.BlockSpec((S, KC), lambda b: (b, 0)),
            pl.BlockSpec((S, KC), lambda b: (b, 0)),
            pl.BlockSpec((S, QRC), lambda b: (b, 0)),
            pl.BlockSpec((S, Dr), lambda b: (b, 0)),
        ],
        out_specs=pl.BlockSpec((S, KC), lambda b: (b, 0)),
    )(q, kj, vj, qr, kr)


RH = RC // 2


def _outar_body(q_ref, kj_ref, vj_ref, qr_ref, kr_ref, wo_ref, out_ref,
                buf_r, buf_l,
                rs_r_s, rs_r_r, rs_l_s, rs_l_r,
                ag_s, ag_r):
    i = lax.axis_index("i")
    left = (i - 1) % N
    right = (i + 1) % N
    bsem = pltpu.get_barrier_semaphore()
    for nbr in (left, right):
        pl.semaphore_signal(bsem, inc=1, device_id=(nbr,),
                            device_id_type=pl.DeviceIdType.MESH)
    pl.semaphore_wait(bsem, 2)

    wo = wo_ref[...]
    nums = (((1,), (1,)), ((), ()))

    def att_rows(sub):
        b0 = (sub // (S // RH)) * S
        qs = q_ref[pl.ds(sub * RH, RH), :]
        qrs = qr_ref[pl.ds(sub * RH, RH), :]
        kb = kj_ref[pl.ds(b0, S), :]
        vb = vj_ref[pl.ds(b0, S), :]
        krb = kr_ref[pl.ds(b0, S), :]
        ohs = []
        for h in range(HL):
            hd = slice(h * Dh, (h + 1) * Dh)
            hr = slice(h * Dr, (h + 1) * Dr)
            s = (lax.dot_general(qs[:, hd], kb[:, hd], nums,
                                 preferred_element_type=jnp.float32)
                 + lax.dot_general(qrs[:, hr], krb, nums,
                                   preferred_element_type=jnp.float32)) * SCALE
            m = jnp.max(s, axis=-1, keepdims=True)
            e = jnp.exp(s - m)
            p = (e / jnp.sum(e, axis=-1, keepdims=True)).astype(BF)
            ohs.append(jnp.dot(p, vb[:, hd],
                               preferred_element_type=jnp.float32).astype(BF))
        return jnp.concatenate(ohs, axis=1)

    def part2(sub_r, sub_l):
        o2 = jnp.concatenate([att_rows(sub_r), att_rows(sub_l)], axis=0)
        g = jnp.dot(o2, wo, preferred_element_type=jnp.float32)
        return g[:RH], g[RH:]

    pending = []
    CH = D // 2
    sub_t = 2 * ((i + 1) % N)
    sub_b = 2 * ((i - 1) % N) + 1

    def rs_start(buf, ch, hop, dev, ssem, rsem):
        src = N - 1 if hop == 0 else hop - 1
        sl = pl.ds(ch * CH, CH)
        rd = pltpu.make_async_remote_copy(
            src_ref=buf.at[src, :, sl], dst_ref=buf.at[hop, :, sl],
            send_sem=ssem.at[ch, hop], recv_sem=rsem.at[ch, hop],
            device_id=(dev,), device_id_type=pl.DeviceIdType.MESH)
        rd.start()
        pending.append(rd)
        return rd

    own_r, own_l = part2(2 * i, 2 * i + 1)
    buf_r[N - 1] = own_r.astype(BF)
    buf_l[N - 1] = own_l.astype(BF)
    rx = rs_start(buf_r, 0, 0, right, rs_r_s, rs_r_r)
    lx = rs_start(buf_l, 0, 0, left, rs_l_s, rs_l_r)
    ry = rs_start(buf_r, 1, 0, right, rs_r_s, rs_r_r)
    ly = rs_start(buf_l, 1, 0, left, rs_l_s, rs_l_r)
    nxt_r, nxt_l = part2(2 * ((i - 1) % N), 2 * ((i + 1) % N) + 1)
    for s in range(N - 1):
        rx.wait_recv()
        lx.wait_recv()
        if s < N - 2:
            buf_r[s, :, :CH] = (buf_r[s, :, :CH] + nxt_r[:, :CH]).astype(BF)
            buf_l[s, :, :CH] = (buf_l[s, :, :CH] + nxt_l[:, :CH]).astype(BF)
            rx = rs_start(buf_r, 0, s + 1, right, rs_r_s, rs_r_r)
            lx = rs_start(buf_l, 0, s + 1, left, rs_l_s, rs_l_r)
            nnr, nnl = part2(2 * ((i - s - 2) % N),
                             2 * ((i + s + 2) % N) + 1)
        else:
            out_ref[sub_t, :, :CH] = (buf_r[s, :, :CH]
                                      + nxt_r[:, :CH]).astype(BF)
            out_ref[sub_b, :, :CH] = (buf_l[s, :, :CH]
                                      + nxt_l[:, :CH]).astype(BF)
        ry.wait_recv()
        ly.wait_recv()
        if s < N - 2:
            buf_r[s, :, CH:] = (buf_r[s, :, CH:] + nxt_r[:, CH:]).astype(BF)
            buf_l[s, :, CH:] = (buf_l[s, :, CH:] + nxt_l[:, CH:]).astype(BF)
            ry = rs_start(buf_r, 1, s + 1, right, rs_r_s, rs_r_r)
            ly = rs_start(buf_l, 1, s + 1, left, rs_l_s, rs_l_r)
            nxt_r, nxt_l = nnr, nnl
        else:
            out_ref[sub_t, :, CH:] = (buf_r[s, :, CH:]
                                      + nxt_r[:, CH:]).astype(BF)
            out_ref[sub_b, :, CH:] = (buf_l[s, :, CH:]
                                      + nxt_l[:, CH:]).astype(BF)

    def PX(d):
        return d ^ 1

    def PY(d):
        return 4 * (d // 4) + (3 - d % 4)

    def PZ(d):
        return d ^ 4

    tstarts = (0, 1280, 2688)
    twidths = (1280, 1408, 1408)
    orders = ((PX, PY, PZ), (PY, PZ, PX), (PZ, PX, PY))
    held = [[i], [i], [i]]
    ctr = [0, 0, 0]
    for p in range(3):
        phase_descs = []
        for t in range(3):
            P = orders[t][p]
            prt = P(i)
            for d in held[t]:
                for sub in (2 * ((d + 1) % N), 2 * ((d - 1) % N) + 1):
                    piece = out_ref.at[sub, :, pl.ds(tstarts[t], twidths[t])]
                    rd = pltpu.make_async_remote_copy(
                        src_ref=piece, dst_ref=piece,
                        send_sem=ag_s.at[t, ctr[t]],
                        recv_sem=ag_r.at[t, ctr[t]],
                        device_id=(prt,),
                        device_id_type=pl.DeviceIdType.MESH)
                    rd.start()
                    ctr[t] += 1
                    phase_descs.append(rd)
                    pending.append(rd)
            held[t] = held[t] + [P(d) for d in held[t]]
        for rd in phase_descs:
            rd.wait_recv()

    for rd in pending:
        rd.wait_send()


def _outar(q, kj, vj, qr, kr, wo_j):
    return pl.pallas_call(
        _outar_body,
        out_shape=jax.ShapeDtypeStruct((2 * N, RH, D), BF),
        in_specs=[pl.BlockSpec(memory_space=VMEM)] * 6,
        out_specs=pl.BlockSpec(memory_space=VMEM),
        scratch_shapes=[
            VMEM((N, RH, D), BF),
            VMEM((N, RH, D), BF),
            pltpu.SemaphoreType.DMA((2, N - 1)),
            pltpu.SemaphoreType.DMA((2, N - 1)),
            pltpu.SemaphoreType.DMA((2, N - 1)),
            pltpu.SemaphoreType.DMA((2, N - 1)),
            pltpu.SemaphoreType.DMA((3, 14)),
            pltpu.SemaphoreType.DMA((3, 14)),
        ],
        compiler_params=pltpu.CompilerParams(collective_id=1),
    )(q, kj, vj, qr, kr, wo_j)


def kernel(x, Wdkv, Wuk, Wuv, Wq, Wqr, Wkr, Wo):
    i = lax.axis_index("i")
    blk = (i + 1) % N

    xb = x.reshape(M, D).astype(BF)
    dc = Wdkv.shape[1]
    wukb = Wuk.astype(BF).reshape(dc, N, KC).transpose(1, 0, 2)
    wuvb = Wuv.astype(BF).reshape(dc, N, KC).transpose(1, 0, 2)

    wq_j = lax.dynamic_slice(Wq, (0, blk * KC), (D, KC)).astype(BF)
    wqr_j = lax.dynamic_slice(Wqr, (0, blk * QRC), (D, QRC)).astype(BF)
    wo_j = lax.dynamic_slice(Wo, (blk * KC, 0), (KC, D)).astype(BF)
    q, qr, kr, kj, vj = _kvq(xb, Wdkv.astype(BF), wukb, wuvb,
                             wq_j, wqr_j, Wkr.astype(BF))

    out16 = _outar(q, kj, vj, qr, kr, wo_j)
    return out16.reshape(M, D).reshape(B, S, D)
